# Initial kernel scaffold; baseline (speedup 1.0000x reference)
import jax
import jax.numpy as jnp
from jax.experimental import pallas as pl


def kernel(x, edge_index, W1, b1, W2, b2, W3, b3, Wp1, bp1, Wp2, bp2):
    raise NotImplementedError("write your pallas kernel here")


# speedup vs baseline: 2.2103x; 2.2103x over previous
import functools

import jax
import jax.numpy as jnp
from jax.experimental import pallas as pl

N = 100000
BLK = 2000


def _row_spec(f):
    return pl.BlockSpec((BLK, f), lambda i: (i, 0))


def _full_spec(shape):
    return pl.BlockSpec(shape, lambda i: tuple(0 for _ in shape))


def _tc_call(body, out_f, in_arrays, in_specs):
    return pl.pallas_call(
        body,
        grid=(N // BLK,),
        in_specs=in_specs,
        out_specs=_row_spec(out_f),
        out_shape=jax.ShapeDtypeStruct((N, out_f), jnp.float32),
    )(*in_arrays)


def _prep_body(deg_ref, x_ref, r_ref, u0_ref):
    r = jax.lax.rsqrt(jnp.maximum(deg_ref[...], 1e-12))
    r_ref[...] = r
    u0_ref[...] = r * x_ref[...]


def _tc_prep(deg, x):
    out = pl.pallas_call(
        _prep_body,
        grid=(N // BLK,),
        in_specs=[_row_spec(1), _row_spec(2)],
        out_specs=(_row_spec(1), _row_spec(2)),
        out_shape=(
            jax.ShapeDtypeStruct((N, 1), jnp.float32),
            jax.ShapeDtypeStruct((N, 2), jnp.float32),
        ),
    )(deg, x)
    return out


def _tcA_body(agg0_ref, u0_ref, r_ref, W1_ref, b1_ref, u1_ref):
    r = r_ref[...]
    s = r * (agg0_ref[...] + u0_ref[...])
    h = jnp.dot(s, W1_ref[...], preferred_element_type=jnp.float32)
    h = jnp.maximum(h + b1_ref[...], 0.0)
    u1_ref[...] = r * h


def _tcB_body(agg1_ref, u1_ref, r_ref, W2_ref, b2_ref, W3_ref, g_ref):
    r = r_ref[...]
    s = r * (agg1_ref[...] + u1_ref[...])
    h = jnp.dot(s, W2_ref[...], preferred_element_type=jnp.float32)
    h = jnp.maximum(h + b2_ref[...], 0.0)
    g_ref[...] = jnp.dot(r * h, W3_ref[...], preferred_element_type=jnp.float32)


def _tcC_body(aggg_ref, g_ref, r_ref, b3_ref, Wp1_ref, bp1_ref, Wp2_ref,
              bp2_ref, o_ref):
    r = r_ref[...]
    h3 = jnp.maximum(r * (aggg_ref[...] + g_ref[...]) + b3_ref[...], 0.0)
    p = jnp.dot(h3, Wp1_ref[...], preferred_element_type=jnp.float32)
    p = jnp.maximum(p + bp1_ref[...], 0.0)
    o = jnp.dot(p, Wp2_ref[...], preferred_element_type=jnp.float32)
    o_ref[...] = jax.nn.sigmoid(o + bp2_ref[...])


def _agg(u, src, dst):
    return jnp.zeros_like(u).at[dst].add(u[src], mode="drop")


def kernel(x, edge_index, W1, b1, W2, b2, W3, b3, Wp1, bp1, Wp2, bp2):
    src = edge_index[0].astype(jnp.int32)
    dst = edge_index[1].astype(jnp.int32)

    deg = (jnp.zeros((N,), jnp.float32).at[dst].add(1.0) + 1.0).reshape(N, 1)
    r, u0 = _tc_prep(deg, x)

    agg0 = _agg(u0, src, dst)
    u1 = _tc_call(
        _tcA_body, 256,
        (agg0, u0, r, W1, b1.reshape(1, 256)),
        [_row_spec(2), _row_spec(2), _row_spec(1), _full_spec((2, 256)),
         _full_spec((1, 256))],
    )

    agg1 = _agg(u1, src, dst)
    g = _tc_call(
        _tcB_body, 128,
        (agg1, u1, r, W2, b2.reshape(1, 256), W3),
        [_row_spec(256), _row_spec(256), _row_spec(1), _full_spec((256, 256)),
         _full_spec((1, 256)), _full_spec((256, 128))],
    )

    aggg = _agg(g, src, dst)
    out = _tc_call(
        _tcC_body, 1,
        (aggg, g, r, b3.reshape(1, 128), Wp1, bp1.reshape(1, 32), Wp2,
         bp2.reshape(1, 1)),
        [_row_spec(128), _row_spec(128), _row_spec(1), _full_spec((1, 128)),
         _full_spec((128, 32)), _full_spec((1, 32)), _full_spec((32, 1)),
         _full_spec((1, 1))],
    )
    return out


# speedup vs baseline: 2.7269x; 1.2337x over previous
import functools

import jax
import jax.numpy as jnp
from jax import lax
from jax.experimental import pallas as pl
from jax.experimental.pallas import tpu as pltpu
from jax.experimental.pallas import tpu_sc as plsc

N = 100000
E = 1600000
BLK = 2000

RPN = 3136
NPAD = 32 * RPN
CH_D = 8000
CH_A = 4000
HALF = 50000

_MESH = dict(core_axis_name="c", subcore_axis_name="s")
_CP = pltpu.CompilerParams(needs_layout_passes=False)


@functools.partial(
    pl.kernel,
    out_type=jax.ShapeDtypeStruct((NPAD,), jnp.float32),
    mesh=plsc.VectorSubcoreMesh(**_MESH),
    scratch_types=[
        pltpu.VMEM((CH_D,), jnp.int32),
        pltpu.VMEM((RPN,), jnp.float32),
    ],
    compiler_params=_CP,
)
def _sc_deg(dst_hbm, out_hbm, dchunk, acc):
    c = lax.axis_index("c")
    s = lax.axis_index("s")
    w = s * 2 + c
    lo = w * RPN
    zero16 = jnp.zeros((16,), jnp.float32)
    ones16 = jnp.ones((16,), jnp.float32)

    def zero_body(j, carry):
        acc[pl.ds(j * 16, 16)] = zero16
        return carry

    lax.fori_loop(0, RPN // 16, zero_body, 0)

    def chunk_body(kk, carry):
        pltpu.sync_copy(dst_hbm.at[pl.ds(kk * CH_D, CH_D)], dchunk)

        def vec_body(j, c2):
            dv = dchunk[pl.ds(j * 16, 16)]
            ldv = dv - lo
            m = (ldv >= 0) & (ldv < RPN)
            plsc.addupdate_scatter(acc, [ldv], ones16, mask=m)
            return c2

        lax.fori_loop(0, CH_D // 16, vec_body, 0)
        return carry

    lax.fori_loop(0, E // CH_D, chunk_body, 0)
    pltpu.sync_copy(acc, out_hbm.at[pl.ds(lo, RPN)])


@functools.partial(
    pl.kernel,
    out_type=jax.ShapeDtypeStruct((2 * NPAD,), jnp.float32),
    mesh=plsc.VectorSubcoreMesh(**_MESH),
    scratch_types=[
        pltpu.VMEM((CH_A,), jnp.int32),
        pltpu.VMEM((CH_A,), jnp.int32),
        pltpu.VMEM((2 * HALF,), jnp.float32),
        pltpu.VMEM((2 * RPN,), jnp.float32),
    ],
    compiler_params=_CP,
)
def _sc_agg2(u_hbm, src_hbm, dst_hbm, out_hbm, schunk, dchunk, table, acc):
    c = lax.axis_index("c")
    s = lax.axis_index("s")
    w = s * 2 + c
    lo = w * RPN
    zero16 = jnp.zeros((16,), jnp.float32)

    def zero_body(j, carry):
        acc[pl.ds(j * 16, 16)] = zero16
        return carry

    lax.fori_loop(0, 2 * RPN // 16, zero_body, 0)

    for p in range(2):
        pltpu.sync_copy(u_hbm.at[pl.ds(p * 2 * HALF, 2 * HALF)], table)

        def chunk_body(kk, carry):
            pltpu.sync_copy(src_hbm.at[pl.ds(kk * CH_A, CH_A)], schunk)
            pltpu.sync_copy(dst_hbm.at[pl.ds(kk * CH_A, CH_A)], dchunk)

            def vec_body(j, c2):
                dv = dchunk[pl.ds(j * 16, 16)]
                sv = schunk[pl.ds(j * 16, 16)]
                ldv = dv - lo
                lsv = sv - p * HALF
                m = ((ldv >= 0) & (ldv < RPN)
                     & (lsv >= 0) & (lsv < HALF))
                g0 = plsc.load_gather(table, [2 * lsv], mask=m)
                g1 = plsc.load_gather(table, [2 * lsv + 1], mask=m)
                i0 = 2 * ldv
                plsc.addupdate_scatter(acc, [i0], g0, mask=m)
                plsc.addupdate_scatter(acc, [i0 + 1], g1, mask=m)
                return c2

            lax.fori_loop(0, CH_A // 16, vec_body, 0)
            return carry

        lax.fori_loop(0, E // CH_A, chunk_body, 0)

    pltpu.sync_copy(acc, out_hbm.at[pl.ds(2 * lo, 2 * RPN)])


def _row_spec(f):
    return pl.BlockSpec((BLK, f), lambda i: (i, 0))


def _full_spec(shape):
    return pl.BlockSpec(shape, lambda i: tuple(0 for _ in shape))


def _tc_call(body, out_shapes, in_arrays, in_specs, out_specs):
    return pl.pallas_call(
        body,
        grid=(N // BLK,),
        in_specs=in_specs,
        out_specs=out_specs,
        out_shape=out_shapes,
    )(*in_arrays)


def _prep_body(d_ref, x_ref, r_ref, u0_ref):
    deg = d_ref[...] + 1.0
    r = lax.rsqrt(jnp.maximum(deg, 1e-12))
    r_ref[...] = r
    u0_ref[...] = r * x_ref[...]


def _tcA_body(a_ref, u0_ref, r_ref, W1_ref, b1_ref, u1_ref):
    r = r_ref[...]
    sH = r * (a_ref[...] + u0_ref[...])
    h = jnp.dot(sH, W1_ref[...], preferred_element_type=jnp.float32)
    h = jnp.maximum(h + b1_ref[...], 0.0)
    u1_ref[...] = r * h


def _tcB_body(agg1_ref, u1_ref, r_ref, W2_ref, b2_ref, W3_ref, g_ref):
    r = r_ref[...]
    sH = r * (agg1_ref[...] + u1_ref[...])
    h = jnp.dot(sH, W2_ref[...], preferred_element_type=jnp.float32)
    h = jnp.maximum(h + b2_ref[...], 0.0)
    g_ref[...] = jnp.dot(r * h, W3_ref[...], preferred_element_type=jnp.float32)


def _tcC_body(aggg_ref, g_ref, r_ref, b3_ref, Wp1_ref, bp1_ref, Wp2_ref,
              bp2_ref, o_ref):
    r = r_ref[...]
    h3 = jnp.maximum(r * (aggg_ref[...] + g_ref[...]) + b3_ref[...], 0.0)
    p = jnp.dot(h3, Wp1_ref[...], preferred_element_type=jnp.float32)
    p = jnp.maximum(p + bp1_ref[...], 0.0)
    o = jnp.dot(p, Wp2_ref[...], preferred_element_type=jnp.float32)
    o_ref[...] = jax.nn.sigmoid(o + bp2_ref[...])


def _agg_wide(u, src, dst):
    return jnp.zeros_like(u).at[dst].add(u[src], mode="drop")


def kernel(x, edge_index, W1, b1, W2, b2, W3, b3, Wp1, bp1, Wp2, bp2):
    src = edge_index[0].astype(jnp.int32)
    dst = edge_index[1].astype(jnp.int32)

    deg = _sc_deg(dst)[:N].reshape(N, 1)
    r, u0 = _tc_call(
        _prep_body,
        (jax.ShapeDtypeStruct((N, 1), jnp.float32),
         jax.ShapeDtypeStruct((N, 2), jnp.float32)),
        (deg, x),
        [_row_spec(1), _row_spec(2)],
        (_row_spec(1), _row_spec(2)),
    )

    agg0 = _sc_agg2(u0.reshape(2 * N), src, dst).reshape(NPAD, 2)[:N]
    u1 = _tc_call(
        _tcA_body, jax.ShapeDtypeStruct((N, 256), jnp.float32),
        (agg0, u0, r, W1, b1.reshape(1, 256)),
        [_row_spec(2), _row_spec(2), _row_spec(1), _full_spec((2, 256)),
         _full_spec((1, 256))],
        _row_spec(256),
    )

    agg1 = _agg_wide(u1, src, dst)
    g = _tc_call(
        _tcB_body, jax.ShapeDtypeStruct((N, 128), jnp.float32),
        (agg1, u1, r, W2, b2.reshape(1, 256), W3),
        [_row_spec(256), _row_spec(256), _row_spec(1), _full_spec((256, 256)),
         _full_spec((1, 256)), _full_spec((256, 128))],
        _row_spec(128),
    )

    agg2 = _agg_wide(g, src, dst)
    out = _tc_call(
        _tcC_body, jax.ShapeDtypeStruct((N, 1), jnp.float32),
        (agg2, g, r, b3.reshape(1, 128), Wp1, bp1.reshape(1, 32), Wp2,
         bp2.reshape(1, 1)),
        [_row_spec(128), _row_spec(128), _row_spec(1), _full_spec((1, 128)),
         _full_spec((128, 32)), _full_spec((1, 32)), _full_spec((32, 1)),
         _full_spec((1, 1))],
        _row_spec(1),
    )
    return out
